# baseline (speedup 1.0000x reference)
import functools

import jax
import jax.numpy as jnp
from jax import lax
from jax.experimental import pallas as pl
from jax.experimental.pallas import tpu as pltpu
from jax.experimental.pallas import tpu_sc as plsc

C = 96
H = 224
HS = 112
N = HS * HS
OB = 256
NOB = N // OB
ICH = 1792
NIC = N // ICH
RS = 32
NSTRIP = H // RS
RSH = RS // 2
PIX = RS * H
SPIX = RSH * HS


CP = 128


def _prep_body(x_ref, mdc_ref, xt_ref, xtr_ref, nrow_ref, ncol_ref):
    x = x_ref[...]
    xt = x.T
    xtr_ref[...] = xt
    xt_ref[...] = jnp.concatenate(
        [xt, jnp.zeros((N, CP - C), jnp.float32)], axis=1)
    nr = jnp.sqrt(jnp.sum(x * x + jnp.float32(1e-6), axis=0, keepdims=True))
    nrow_ref[...] = nr
    md = mdc_ref[...]
    ncol_ref[...] = jnp.where(md != 0.0, nr.T, jnp.float32(jnp.nan))


def _prep(fd, mdc):
    return pl.pallas_call(
        _prep_body,
        out_shape=[
            jax.ShapeDtypeStruct((N, CP), jnp.float32),
            jax.ShapeDtypeStruct((N, C), jnp.float32),
            jax.ShapeDtypeStruct((1, N), jnp.float32),
            jax.ShapeDtypeStruct((N, 1), jnp.float32),
        ],
    )(fd, mdc)


MR = 64


def _sim_body(xt_ref, xo_ref, nrow_ref, ncol_ref, mdo_ref, idx_ref, s_ref):
    ob = pl.program_id(0)
    xo = xo_ref[...]
    nob = jnp.broadcast_to(nrow_ref[...], (MR, OB))

    rm = jnp.full((MR, OB), -jnp.inf, jnp.float32)
    ri = jnp.zeros((MR, OB), jnp.int32)
    for k in range(NIC):
        a = xt_ref[pl.ds(k * ICH, ICH), :]
        s_ref[...] = lax.dot_general(a, xo, (((1,), (0,)), ((), ())),
                                     preferred_element_type=jnp.float32)
        for j in range(ICH // MR):
            base = k * ICH + j * MR
            ss = s_ref[j * MR:(j + 1) * MR, :]
            nc = ncol_ref[pl.ds(base, MR), :]
            d = ss / (nc * nob)
            better = d > rm
            rm = jnp.where(better, d, rm)
            ri = jnp.where(better, jnp.int32(base // MR), ri)

    rows = ri * MR + lax.broadcasted_iota(jnp.int32, (MR, OB), 0)
    cmax = jnp.max(rm, axis=0, keepdims=True)
    runidx = jnp.min(jnp.where(rm == cmax, rows, jnp.int32(2**30)),
                     axis=0, keepdims=True)
    og = ob * OB + lax.broadcasted_iota(jnp.int32, (1, OB), 1)
    idx_ref[...] = jnp.where(mdo_ref[...] != 0.0, og, runidx)


def _sim_argmax(xt, fd, nrow, ncol, mdr):
    return pl.pallas_call(
        _sim_body,
        grid=(NOB,),
        in_specs=[
            pl.BlockSpec((N, C), lambda i: (0, 0)),
            pl.BlockSpec((C, OB), lambda i: (0, i)),
            pl.BlockSpec((1, OB), lambda i: (0, i)),
            pl.BlockSpec((N, 1), lambda i: (0, 0)),
            pl.BlockSpec((1, OB), lambda i: (0, i)),
        ],
        out_specs=pl.BlockSpec((1, OB), lambda i: (0, i)),
        out_shape=jax.ShapeDtypeStruct((1, N), jnp.int32),
        scratch_shapes=[pltpu.VMEM((ICH, OB), jnp.float32)],
    )(xt, fd, nrow, ncol, mdr)


def _sc_gather(table, idx):
    info = plsc.get_sparse_core_info()
    nc, ns = info.num_cores, info.num_subcores
    bpw = N // (nc * ns)
    mesh = plsc.VectorSubcoreMesh(core_axis_name="c", subcore_axis_name="s")

    @functools.partial(
        pl.kernel, mesh=mesh,
        out_type=jax.ShapeDtypeStruct((N, CP), jnp.float32),
        scratch_types=[
            pltpu.VMEM((bpw,), jnp.int32),
            pltpu.VMEM((bpw, CP), jnp.float32),
            pltpu.SemaphoreType.DMA,
        ],
    )
    def k(table_hbm, idx_hbm, out_hbm, idx_v, rows_v, sem):
        wid = lax.axis_index("s") * nc + lax.axis_index("c")
        base = wid * bpw
        pltpu.sync_copy(idx_hbm.at[pl.ds(base, bpw)], idx_v)
        copies = []
        for off in range(0, bpw, 128):
            ln = min(128, bpw - off)
            copies.append(pltpu.async_copy(
                table_hbm.at[idx_v.at[pl.ds(off, ln)]],
                rows_v.at[pl.ds(off, ln)], sem))
        for cp in copies:
            cp.wait()
        pltpu.sync_copy(rows_v, out_hbm.at[pl.ds(base, bpw)])

    return k(table, idx)


def _up(pm):
    p4 = pm.reshape(RSH, 1, HS, C)
    u = jnp.concatenate([p4, p4], axis=1).reshape(RS * HS, C)
    t3 = u.T.reshape(C, RS, HS)
    ei = lax.broadcasted_iota(jnp.int32, (HS, H), 0)
    ej = lax.broadcasted_iota(jnp.int32, (HS, H), 1)
    ecol = (ej // 2 == ei).astype(jnp.float32)
    return lax.dot_general(t3, ecol, (((2,), (0,)), ((), ())),
                           preferred_element_type=jnp.float32,
                           precision=lax.Precision.HIGHEST)


def _final_body(f_ref, xsel_ref, w1_ref, w2_ref, b1_ref, b2_ref,
                ffin_ref, fout_ref):
    xsT = xsel_ref[...][:, :C]
    z1T = lax.dot_general(xsT, w1_ref[...], (((1,), (1,)), ((), ())),
                          preferred_element_type=jnp.float32)
    fout_ref[...] = _up(xsT)
    fb = f_ref[...].reshape(C, PIX)
    y2 = lax.dot_general(w2_ref[...], fb, (((1,), (0,)), ((), ())),
                         preferred_element_type=jnp.float32)
    tot = _up(z1T).reshape(C, PIX) + b1_ref[...] + y2 + b2_ref[...]
    ffin_ref[...] = jnp.where(tot >= 0, tot, 0.2 * tot).reshape(C, RS, H)


def _final(f3, xsel, w1m, w2m, b1c, b2c):
    return pl.pallas_call(
        _final_body,
        grid=(NSTRIP,),
        in_specs=[
            pl.BlockSpec((C, RS, H), lambda i: (0, i, 0)),
            pl.BlockSpec((SPIX, CP), lambda i: (i, 0)),
            pl.BlockSpec((C, C), lambda i: (0, 0)),
            pl.BlockSpec((C, C), lambda i: (0, 0)),
            pl.BlockSpec((C, 1), lambda i: (0, 0)),
            pl.BlockSpec((C, 1), lambda i: (0, 0)),
        ],
        out_specs=[
            pl.BlockSpec((C, RS, H), lambda i: (0, i, 0)),
            pl.BlockSpec((C, RS, H), lambda i: (0, i, 0)),
        ],
        out_shape=[
            jax.ShapeDtypeStruct((C, H, H), jnp.float32),
            jax.ShapeDtypeStruct((C, H, H), jnp.float32),
        ],
    )(f3, xsel, w1m, w2m, b1c, b2c)


def kernel(f, mask, w1, b1, w2, b2):
    f3 = f[0]
    fd = f3[:, ::2, ::2].reshape(C, N)
    md = mask[0, 0, ::2, ::2]
    mdc = md.reshape(N, 1)
    mdr = md.reshape(1, N)

    xt, xtr, nrow, ncol = _prep(fd, mdc)
    idx = _sim_argmax(xtr, fd, nrow, ncol, mdr).reshape(N)
    xsel = _sc_gather(xt, idx)
    ffin, fout = _final(f3, xsel, w1[:, :, 0, 0], w2[:, :, 0, 0],
                        b1.reshape(C, 1), b2.reshape(C, 1))
    return (ffin[None], fout[None])


# speedup vs baseline: 1.0141x; 1.0141x over previous
import functools

import jax
import jax.numpy as jnp
from jax import lax
from jax.experimental import pallas as pl
from jax.experimental.pallas import tpu as pltpu
from jax.experimental.pallas import tpu_sc as plsc

C = 96
H = 224
HS = 112
N = HS * HS
OB = 256
NOB = N // OB
ICH = 1792
NIC = N // ICH
RS = 32
NSTRIP = H // RS
RSH = RS // 2
PIX = RS * H
SPIX = RSH * HS


CP = 128


def _prep_body(x_ref, mdc_ref, xt_ref, xtr_ref, nrow_ref, ncol_ref):
    x = x_ref[...]
    xt = x.T
    xtr_ref[...] = xt
    xt_ref[...] = jnp.concatenate(
        [xt, jnp.zeros((N, CP - C), jnp.float32)], axis=1)
    nr = jnp.sqrt(jnp.sum(x * x + jnp.float32(1e-6), axis=0, keepdims=True))
    nrow_ref[...] = nr
    md = mdc_ref[...]
    ncol_ref[...] = jnp.where(md != 0.0, nr.T, jnp.float32(jnp.nan))


def _prep(fd, mdc):
    return pl.pallas_call(
        _prep_body,
        out_shape=[
            jax.ShapeDtypeStruct((N, CP), jnp.float32),
            jax.ShapeDtypeStruct((N, C), jnp.float32),
            jax.ShapeDtypeStruct((1, N), jnp.float32),
            jax.ShapeDtypeStruct((N, 1), jnp.float32),
        ],
    )(fd, mdc)


MR = 16


def _sim_body(xt_ref, xo_ref, nrow_ref, ncol_ref, mdo_ref, idx_ref, s_ref):
    ob = pl.program_id(0)
    xo = xo_ref[...]
    nob = jnp.broadcast_to(nrow_ref[...], (MR, OB))

    rm = jnp.full((MR, OB), -jnp.inf, jnp.float32)
    ri = jnp.zeros((MR, OB), jnp.int32)
    for k in range(NIC):
        a = xt_ref[pl.ds(k * ICH, ICH), :]
        s_ref[...] = lax.dot_general(a, xo, (((1,), (0,)), ((), ())),
                                     preferred_element_type=jnp.float32)
        for j in range(ICH // MR):
            base = k * ICH + j * MR
            ss = s_ref[j * MR:(j + 1) * MR, :]
            nc = ncol_ref[pl.ds(base, MR), :]
            d = ss / (nc * nob)
            better = d > rm
            rm = jnp.where(better, d, rm)
            ri = jnp.where(better, jnp.int32(base // MR), ri)

    rows = ri * MR + lax.broadcasted_iota(jnp.int32, (MR, OB), 0)
    cmax = jnp.max(rm, axis=0, keepdims=True)
    runidx = jnp.min(jnp.where(rm == cmax, rows, jnp.int32(2**30)),
                     axis=0, keepdims=True)
    og = ob * OB + lax.broadcasted_iota(jnp.int32, (1, OB), 1)
    idx_ref[...] = jnp.where(mdo_ref[...] != 0.0, og, runidx)


def _sim_argmax(xt, fd, nrow, ncol, mdr):
    return pl.pallas_call(
        _sim_body,
        grid=(NOB,),
        in_specs=[
            pl.BlockSpec((N, C), lambda i: (0, 0)),
            pl.BlockSpec((C, OB), lambda i: (0, i)),
            pl.BlockSpec((1, OB), lambda i: (0, i)),
            pl.BlockSpec((N, 1), lambda i: (0, 0)),
            pl.BlockSpec((1, OB), lambda i: (0, i)),
        ],
        out_specs=pl.BlockSpec((1, OB), lambda i: (0, i)),
        out_shape=jax.ShapeDtypeStruct((1, N), jnp.int32),
        scratch_shapes=[pltpu.VMEM((ICH, OB), jnp.float32)],
    )(xt, fd, nrow, ncol, mdr)


def _sc_gather(table, idx):
    info = plsc.get_sparse_core_info()
    nc, ns = info.num_cores, info.num_subcores
    bpw = N // (nc * ns)
    mesh = plsc.VectorSubcoreMesh(core_axis_name="c", subcore_axis_name="s")

    @functools.partial(
        pl.kernel, mesh=mesh,
        out_type=jax.ShapeDtypeStruct((N, CP), jnp.float32),
        scratch_types=[
            pltpu.VMEM((bpw,), jnp.int32),
            pltpu.VMEM((bpw, CP), jnp.float32),
            pltpu.SemaphoreType.DMA,
        ],
    )
    def k(table_hbm, idx_hbm, out_hbm, idx_v, rows_v, sem):
        wid = lax.axis_index("s") * nc + lax.axis_index("c")
        base = wid * bpw
        pltpu.sync_copy(idx_hbm.at[pl.ds(base, bpw)], idx_v)
        copies = []
        for off in range(0, bpw, 128):
            ln = min(128, bpw - off)
            copies.append(pltpu.async_copy(
                table_hbm.at[idx_v.at[pl.ds(off, ln)]],
                rows_v.at[pl.ds(off, ln)], sem))
        for cp in copies:
            cp.wait()
        pltpu.sync_copy(rows_v, out_hbm.at[pl.ds(base, bpw)])

    return k(table, idx)


def _up(pm):
    p4 = pm.reshape(RSH, 1, HS, C)
    u = jnp.concatenate([p4, p4], axis=1).reshape(RS * HS, C)
    t3 = u.T.reshape(C, RS, HS)
    ei = lax.broadcasted_iota(jnp.int32, (HS, H), 0)
    ej = lax.broadcasted_iota(jnp.int32, (HS, H), 1)
    ecol = (ej // 2 == ei).astype(jnp.float32)
    return lax.dot_general(t3, ecol, (((2,), (0,)), ((), ())),
                           preferred_element_type=jnp.float32,
                           precision=lax.Precision.HIGHEST)


def _final_body(f_ref, xsel_ref, w1_ref, w2_ref, b1_ref, b2_ref,
                ffin_ref, fout_ref):
    xsT = xsel_ref[...][:, :C]
    z1T = lax.dot_general(xsT, w1_ref[...], (((1,), (1,)), ((), ())),
                          preferred_element_type=jnp.float32)
    fout_ref[...] = _up(xsT)
    fb = f_ref[...].reshape(C, PIX)
    y2 = lax.dot_general(w2_ref[...], fb, (((1,), (0,)), ((), ())),
                         preferred_element_type=jnp.float32)
    tot = _up(z1T).reshape(C, PIX) + b1_ref[...] + y2 + b2_ref[...]
    ffin_ref[...] = jnp.where(tot >= 0, tot, 0.2 * tot).reshape(C, RS, H)


def _final(f3, xsel, w1m, w2m, b1c, b2c):
    return pl.pallas_call(
        _final_body,
        grid=(NSTRIP,),
        in_specs=[
            pl.BlockSpec((C, RS, H), lambda i: (0, i, 0)),
            pl.BlockSpec((SPIX, CP), lambda i: (i, 0)),
            pl.BlockSpec((C, C), lambda i: (0, 0)),
            pl.BlockSpec((C, C), lambda i: (0, 0)),
            pl.BlockSpec((C, 1), lambda i: (0, 0)),
            pl.BlockSpec((C, 1), lambda i: (0, 0)),
        ],
        out_specs=[
            pl.BlockSpec((C, RS, H), lambda i: (0, i, 0)),
            pl.BlockSpec((C, RS, H), lambda i: (0, i, 0)),
        ],
        out_shape=[
            jax.ShapeDtypeStruct((C, H, H), jnp.float32),
            jax.ShapeDtypeStruct((C, H, H), jnp.float32),
        ],
    )(f3, xsel, w1m, w2m, b1c, b2c)


def kernel(f, mask, w1, b1, w2, b2):
    f3 = f[0]
    fd = f3[:, ::2, ::2].reshape(C, N)
    md = mask[0, 0, ::2, ::2]
    mdc = md.reshape(N, 1)
    mdr = md.reshape(1, N)

    xt, xtr, nrow, ncol = _prep(fd, mdc)
    idx = _sim_argmax(xtr, fd, nrow, ncol, mdr).reshape(N)
    xsel = _sc_gather(xt, idx)
    ffin, fout = _final(f3, xsel, w1[:, :, 0, 0], w2[:, :, 0, 0],
                        b1.reshape(C, 1), b2.reshape(C, 1))
    return (ffin[None], fout[None])


# speedup vs baseline: 1.0175x; 1.0034x over previous
import functools

import jax
import jax.numpy as jnp
from jax import lax
from jax.experimental import pallas as pl
from jax.experimental.pallas import tpu as pltpu
from jax.experimental.pallas import tpu_sc as plsc

C = 96
H = 224
HS = 112
N = HS * HS
OB = 256
NOB = N // OB
ICH = 3136
NIC = N // ICH
RS = 32
NSTRIP = H // RS
RSH = RS // 2
PIX = RS * H
SPIX = RSH * HS


CP = 128


def _prep_body(x_ref, mdc_ref, xt_ref, xtr_ref, nrow_ref, ncol_ref):
    x = x_ref[...]
    xt = x.T
    xtr_ref[...] = xt
    xt_ref[...] = jnp.concatenate(
        [xt, jnp.zeros((N, CP - C), jnp.float32)], axis=1)
    nr = jnp.sqrt(jnp.sum(x * x + jnp.float32(1e-6), axis=0, keepdims=True))
    nrow_ref[...] = nr
    md = mdc_ref[...]
    ncol_ref[...] = jnp.where(md != 0.0, nr.T, jnp.float32(jnp.nan))


def _prep(fd, mdc):
    return pl.pallas_call(
        _prep_body,
        out_shape=[
            jax.ShapeDtypeStruct((N, CP), jnp.float32),
            jax.ShapeDtypeStruct((N, C), jnp.float32),
            jax.ShapeDtypeStruct((1, N), jnp.float32),
            jax.ShapeDtypeStruct((N, 1), jnp.float32),
        ],
    )(fd, mdc)


MR = 16


def _sim_body(xt_ref, xo_ref, nrow_ref, ncol_ref, mdo_ref, idx_ref, s_ref):
    ob = pl.program_id(0)
    xo = xo_ref[...]
    nob = jnp.broadcast_to(nrow_ref[...], (MR, OB))

    rm = jnp.full((MR, OB), -jnp.inf, jnp.float32)
    ri = jnp.zeros((MR, OB), jnp.int32)
    for k in range(NIC):
        a = xt_ref[pl.ds(k * ICH, ICH), :]
        s_ref[...] = lax.dot_general(a, xo, (((1,), (0,)), ((), ())),
                                     preferred_element_type=jnp.float32)
        for j in range(ICH // MR):
            base = k * ICH + j * MR
            ss = s_ref[j * MR:(j + 1) * MR, :]
            nc = ncol_ref[pl.ds(base, MR), :]
            d = ss / (nc * nob)
            better = d > rm
            rm = jnp.where(better, d, rm)
            ri = jnp.where(better, jnp.int32(base // MR), ri)

    rows = ri * MR + lax.broadcasted_iota(jnp.int32, (MR, OB), 0)
    cmax = jnp.max(rm, axis=0, keepdims=True)
    runidx = jnp.min(jnp.where(rm == cmax, rows, jnp.int32(2**30)),
                     axis=0, keepdims=True)
    og = ob * OB + lax.broadcasted_iota(jnp.int32, (1, OB), 1)
    idx_ref[...] = jnp.where(mdo_ref[...] != 0.0, og, runidx)


def _sim_argmax(xt, fd, nrow, ncol, mdr):
    return pl.pallas_call(
        _sim_body,
        grid=(NOB,),
        in_specs=[
            pl.BlockSpec((N, C), lambda i: (0, 0)),
            pl.BlockSpec((C, OB), lambda i: (0, i)),
            pl.BlockSpec((1, OB), lambda i: (0, i)),
            pl.BlockSpec((N, 1), lambda i: (0, 0)),
            pl.BlockSpec((1, OB), lambda i: (0, i)),
        ],
        out_specs=pl.BlockSpec((1, OB), lambda i: (0, i)),
        out_shape=jax.ShapeDtypeStruct((1, N), jnp.int32),
        scratch_shapes=[pltpu.VMEM((ICH, OB), jnp.float32)],
    )(xt, fd, nrow, ncol, mdr)


def _sc_gather(table, idx):
    info = plsc.get_sparse_core_info()
    nc, ns = info.num_cores, info.num_subcores
    bpw = N // (nc * ns)
    mesh = plsc.VectorSubcoreMesh(core_axis_name="c", subcore_axis_name="s")

    @functools.partial(
        pl.kernel, mesh=mesh,
        out_type=jax.ShapeDtypeStruct((N, CP), jnp.float32),
        scratch_types=[
            pltpu.VMEM((bpw,), jnp.int32),
            pltpu.VMEM((bpw, CP), jnp.float32),
            pltpu.SemaphoreType.DMA,
        ],
    )
    def k(table_hbm, idx_hbm, out_hbm, idx_v, rows_v, sem):
        wid = lax.axis_index("s") * nc + lax.axis_index("c")
        base = wid * bpw
        pltpu.sync_copy(idx_hbm.at[pl.ds(base, bpw)], idx_v)
        copies = []
        for off in range(0, bpw, 128):
            ln = min(128, bpw - off)
            copies.append(pltpu.async_copy(
                table_hbm.at[idx_v.at[pl.ds(off, ln)]],
                rows_v.at[pl.ds(off, ln)], sem))
        for cp in copies:
            cp.wait()
        pltpu.sync_copy(rows_v, out_hbm.at[pl.ds(base, bpw)])

    return k(table, idx)


def _up(pm):
    p4 = pm.reshape(RSH, 1, HS, C)
    u = jnp.concatenate([p4, p4], axis=1).reshape(RS * HS, C)
    t3 = u.T.reshape(C, RS, HS)
    ei = lax.broadcasted_iota(jnp.int32, (HS, H), 0)
    ej = lax.broadcasted_iota(jnp.int32, (HS, H), 1)
    ecol = (ej // 2 == ei).astype(jnp.float32)
    return lax.dot_general(t3, ecol, (((2,), (0,)), ((), ())),
                           preferred_element_type=jnp.float32,
                           precision=lax.Precision.HIGHEST)


def _final_body(f_ref, xsel_ref, w1_ref, w2_ref, b1_ref, b2_ref,
                ffin_ref, fout_ref):
    xsT = xsel_ref[...][:, :C]
    z1T = lax.dot_general(xsT, w1_ref[...], (((1,), (1,)), ((), ())),
                          preferred_element_type=jnp.float32)
    fout_ref[...] = _up(xsT)
    fb = f_ref[...].reshape(C, PIX)
    y2 = lax.dot_general(w2_ref[...], fb, (((1,), (0,)), ((), ())),
                         preferred_element_type=jnp.float32)
    tot = _up(z1T).reshape(C, PIX) + b1_ref[...] + y2 + b2_ref[...]
    ffin_ref[...] = jnp.where(tot >= 0, tot, 0.2 * tot).reshape(C, RS, H)


def _final(f3, xsel, w1m, w2m, b1c, b2c):
    return pl.pallas_call(
        _final_body,
        grid=(NSTRIP,),
        in_specs=[
            pl.BlockSpec((C, RS, H), lambda i: (0, i, 0)),
            pl.BlockSpec((SPIX, CP), lambda i: (i, 0)),
            pl.BlockSpec((C, C), lambda i: (0, 0)),
            pl.BlockSpec((C, C), lambda i: (0, 0)),
            pl.BlockSpec((C, 1), lambda i: (0, 0)),
            pl.BlockSpec((C, 1), lambda i: (0, 0)),
        ],
        out_specs=[
            pl.BlockSpec((C, RS, H), lambda i: (0, i, 0)),
            pl.BlockSpec((C, RS, H), lambda i: (0, i, 0)),
        ],
        out_shape=[
            jax.ShapeDtypeStruct((C, H, H), jnp.float32),
            jax.ShapeDtypeStruct((C, H, H), jnp.float32),
        ],
    )(f3, xsel, w1m, w2m, b1c, b2c)


def kernel(f, mask, w1, b1, w2, b2):
    f3 = f[0]
    fd = f3[:, ::2, ::2].reshape(C, N)
    md = mask[0, 0, ::2, ::2]
    mdc = md.reshape(N, 1)
    mdr = md.reshape(1, N)

    xt, xtr, nrow, ncol = _prep(fd, mdc)
    idx = _sim_argmax(xtr, fd, nrow, ncol, mdr).reshape(N)
    xsel = _sc_gather(xt, idx)
    ffin, fout = _final(f3, xsel, w1[:, :, 0, 0], w2[:, :, 0, 0],
                        b1.reshape(C, 1), b2.reshape(C, 1))
    return (ffin[None], fout[None])


# speedup vs baseline: 1.0181x; 1.0005x over previous
import functools

import jax
import jax.numpy as jnp
from jax import lax
from jax.experimental import pallas as pl
from jax.experimental.pallas import tpu as pltpu
from jax.experimental.pallas import tpu_sc as plsc

C = 96
H = 224
HS = 112
N = HS * HS
OB = 256
NOB = N // OB
ICH = 3136
NIC = N // ICH
RS = 32
NSTRIP = H // RS
RSH = RS // 2
PIX = RS * H
SPIX = RSH * HS


CP = 128


def _prep_body(x_ref, mdc_ref, xt_ref, xtr_ref, nrow_ref, ncol_ref):
    x = x_ref[...]
    xt = x.T
    xtr_ref[...] = xt
    xt_ref[...] = jnp.concatenate(
        [xt, jnp.zeros((N, CP - C), jnp.float32)], axis=1)
    nr = jnp.sqrt(jnp.sum(x * x + jnp.float32(1e-6), axis=0, keepdims=True))
    nrow_ref[...] = nr
    md = mdc_ref[...]
    ncol_ref[...] = jnp.where(md != 0.0, nr.T, jnp.float32(jnp.nan))


def _prep(fd, mdc):
    return pl.pallas_call(
        _prep_body,
        out_shape=[
            jax.ShapeDtypeStruct((N, CP), jnp.float32),
            jax.ShapeDtypeStruct((N, C), jnp.float32),
            jax.ShapeDtypeStruct((1, N), jnp.float32),
            jax.ShapeDtypeStruct((N, 1), jnp.float32),
        ],
    )(fd, mdc)


MR = 16


def _sim_body(xt_ref, xo_ref, nrow_ref, ncol_ref, mdo_ref, idx_ref,
              s0_ref, s1_ref):
    ob = pl.program_id(0)
    xo = xo_ref[...]
    nob = jnp.broadcast_to(nrow_ref[...], (MR, OB))

    rm = jnp.full((MR, OB), -jnp.inf, jnp.float32)
    ri = jnp.zeros((MR, OB), jnp.int32)
    for k in range(NIC):
        s_ref = s0_ref if k % 2 == 0 else s1_ref
        a = xt_ref[pl.ds(k * ICH, ICH), :]
        s_ref[...] = lax.dot_general(a, xo, (((1,), (0,)), ((), ())),
                                     preferred_element_type=jnp.float32)
        for j in range(ICH // MR):
            base = k * ICH + j * MR
            ss = s_ref[j * MR:(j + 1) * MR, :]
            nc = ncol_ref[pl.ds(base, MR), :]
            d = ss / (nc * nob)
            better = d > rm
            rm = jnp.where(better, d, rm)
            ri = jnp.where(better, jnp.int32(base // MR), ri)

    rows = ri * MR + lax.broadcasted_iota(jnp.int32, (MR, OB), 0)
    cmax = jnp.max(rm, axis=0, keepdims=True)
    runidx = jnp.min(jnp.where(rm == cmax, rows, jnp.int32(2**30)),
                     axis=0, keepdims=True)
    og = ob * OB + lax.broadcasted_iota(jnp.int32, (1, OB), 1)
    idx_ref[...] = jnp.where(mdo_ref[...] != 0.0, og, runidx)


def _sim_argmax(xt, fd, nrow, ncol, mdr):
    return pl.pallas_call(
        _sim_body,
        grid=(NOB,),
        in_specs=[
            pl.BlockSpec((N, C), lambda i: (0, 0)),
            pl.BlockSpec((C, OB), lambda i: (0, i)),
            pl.BlockSpec((1, OB), lambda i: (0, i)),
            pl.BlockSpec((N, 1), lambda i: (0, 0)),
            pl.BlockSpec((1, OB), lambda i: (0, i)),
        ],
        out_specs=pl.BlockSpec((1, OB), lambda i: (0, i)),
        out_shape=jax.ShapeDtypeStruct((1, N), jnp.int32),
        scratch_shapes=[pltpu.VMEM((ICH, OB), jnp.float32),
                        pltpu.VMEM((ICH, OB), jnp.float32)],
    )(xt, fd, nrow, ncol, mdr)


def _sc_gather(table, idx):
    info = plsc.get_sparse_core_info()
    nc, ns = info.num_cores, info.num_subcores
    bpw = N // (nc * ns)
    mesh = plsc.VectorSubcoreMesh(core_axis_name="c", subcore_axis_name="s")

    @functools.partial(
        pl.kernel, mesh=mesh,
        out_type=jax.ShapeDtypeStruct((N, CP), jnp.float32),
        scratch_types=[
            pltpu.VMEM((bpw,), jnp.int32),
            pltpu.VMEM((bpw, CP), jnp.float32),
            pltpu.SemaphoreType.DMA,
        ],
    )
    def k(table_hbm, idx_hbm, out_hbm, idx_v, rows_v, sem):
        wid = lax.axis_index("s") * nc + lax.axis_index("c")
        base = wid * bpw
        pltpu.sync_copy(idx_hbm.at[pl.ds(base, bpw)], idx_v)
        copies = []
        for off in range(0, bpw, 128):
            ln = min(128, bpw - off)
            copies.append(pltpu.async_copy(
                table_hbm.at[idx_v.at[pl.ds(off, ln)]],
                rows_v.at[pl.ds(off, ln)], sem))
        for cp in copies:
            cp.wait()
        pltpu.sync_copy(rows_v, out_hbm.at[pl.ds(base, bpw)])

    return k(table, idx)


def _up(pm):
    p4 = pm.reshape(RSH, 1, HS, C)
    u = jnp.concatenate([p4, p4], axis=1).reshape(RS * HS, C)
    t3 = u.T.reshape(C, RS, HS)
    ei = lax.broadcasted_iota(jnp.int32, (HS, H), 0)
    ej = lax.broadcasted_iota(jnp.int32, (HS, H), 1)
    ecol = (ej // 2 == ei).astype(jnp.float32)
    return lax.dot_general(t3, ecol, (((2,), (0,)), ((), ())),
                           preferred_element_type=jnp.float32,
                           precision=lax.Precision.HIGHEST)


def _final_body(f_ref, xsel_ref, w1_ref, w2_ref, b1_ref, b2_ref,
                ffin_ref, fout_ref):
    xsT = xsel_ref[...][:, :C]
    z1T = lax.dot_general(xsT, w1_ref[...], (((1,), (1,)), ((), ())),
                          preferred_element_type=jnp.float32)
    fout_ref[...] = _up(xsT)
    fb = f_ref[...].reshape(C, PIX)
    y2 = lax.dot_general(w2_ref[...], fb, (((1,), (0,)), ((), ())),
                         preferred_element_type=jnp.float32)
    tot = _up(z1T).reshape(C, PIX) + b1_ref[...] + y2 + b2_ref[...]
    ffin_ref[...] = jnp.where(tot >= 0, tot, 0.2 * tot).reshape(C, RS, H)


def _final(f3, xsel, w1m, w2m, b1c, b2c):
    return pl.pallas_call(
        _final_body,
        grid=(NSTRIP,),
        in_specs=[
            pl.BlockSpec((C, RS, H), lambda i: (0, i, 0)),
            pl.BlockSpec((SPIX, CP), lambda i: (i, 0)),
            pl.BlockSpec((C, C), lambda i: (0, 0)),
            pl.BlockSpec((C, C), lambda i: (0, 0)),
            pl.BlockSpec((C, 1), lambda i: (0, 0)),
            pl.BlockSpec((C, 1), lambda i: (0, 0)),
        ],
        out_specs=[
            pl.BlockSpec((C, RS, H), lambda i: (0, i, 0)),
            pl.BlockSpec((C, RS, H), lambda i: (0, i, 0)),
        ],
        out_shape=[
            jax.ShapeDtypeStruct((C, H, H), jnp.float32),
            jax.ShapeDtypeStruct((C, H, H), jnp.float32),
        ],
    )(f3, xsel, w1m, w2m, b1c, b2c)


def kernel(f, mask, w1, b1, w2, b2):
    f3 = f[0]
    fd = f3[:, ::2, ::2].reshape(C, N)
    md = mask[0, 0, ::2, ::2]
    mdc = md.reshape(N, 1)
    mdr = md.reshape(1, N)

    xt, xtr, nrow, ncol = _prep(fd, mdc)
    idx = _sim_argmax(xtr, fd, nrow, ncol, mdr).reshape(N)
    xsel = _sc_gather(xt, idx)
    ffin, fout = _final(f3, xsel, w1[:, :, 0, 0], w2[:, :, 0, 0],
                        b1.reshape(C, 1), b2.reshape(C, 1))
    return (ffin[None], fout[None])
